# baseline (speedup 1.0000x reference)
import jax
import jax.numpy as jnp
from jax.experimental import pallas as pl
from jax.experimental.pallas import tpu as pltpu

_SCALE = 8.0


def _scaled_add_kernel(seqs_ref, pos_ref, out_ref):
    out_ref[...] = seqs_ref[...] * _SCALE + pos_ref[...]


def kernel(seqs, pos_emb):
    B, L, D = seqs.shape
    blk = 64
    return pl.pallas_call(
        _scaled_add_kernel,
        grid=(B // blk,),
        in_specs=[
            pl.BlockSpec((blk, L, D), lambda i: (i, 0, 0)),
            pl.BlockSpec((1, L, D), lambda i: (0, 0, 0)),
        ],
        out_specs=pl.BlockSpec((blk, L, D), lambda i: (i, 0, 0)),
        out_shape=jax.ShapeDtypeStruct((B, L, D), jnp.float32),
        compiler_params=pltpu.CompilerParams(
            dimension_semantics=("parallel",),
        ),
    )(seqs, pos_emb[None])


# speedup vs baseline: 1.6688x; 1.6688x over previous
import jax
import jax.numpy as jnp
from jax.experimental import pallas as pl
from jax.experimental.pallas import tpu as pltpu

_SCALE = 8.0


def _scaled_add_kernel(seqs_ref, pos_ref, out_ref):
    out_ref[...] = seqs_ref[...] * _SCALE + pos_ref[...]


def kernel(seqs, pos_emb):
    B, L, D = seqs.shape
    x = seqs.reshape(B, L * D)
    p = pos_emb.reshape(1, L * D)
    blk = 128
    out = pl.pallas_call(
        _scaled_add_kernel,
        grid=(B // blk,),
        in_specs=[
            pl.BlockSpec((blk, L * D), lambda i: (i, 0)),
            pl.BlockSpec((1, L * D), lambda i: (0, 0)),
        ],
        out_specs=pl.BlockSpec((blk, L * D), lambda i: (i, 0)),
        out_shape=jax.ShapeDtypeStruct((B, L * D), jnp.float32),
        compiler_params=pltpu.CompilerParams(
            dimension_semantics=("parallel",),
        ),
    )(x, p)
    return out.reshape(B, L, D)
